# baseline (speedup 1.0000x reference)
import dataclasses

import jax
import jax.numpy as jnp
from jax import lax
from jax.experimental import pallas as pl
from jax.experimental.pallas import tpu as pltpu
from jax.experimental.pallas import tpu_sc as plsc

NC = 2
NS = 16
L = 16
NW = NC * NS

BATCH = 16384
FACTORS = 32
B_PER_W = BATCH // NW


def _make_compiler_params():
    cp = pltpu.CompilerParams()
    fields = pltpu.CompilerParams.__dataclass_fields__
    if "needs_layout_passes" in fields:
        cp = dataclasses.replace(cp, needs_layout_passes=False)
    if "use_tc_tiling_on_sc" in fields:
        cp = dataclasses.replace(cp, use_tc_tiling_on_sc=False)
    return cp


def _mf_dot_kernel(user_hbm, item_hbm, utab_hbm, itab_hbm, out_hbm,
                   uidx_s, iidx_s, ug_v, ig_v, out_v, sem_g, sem_o):
    wid = lax.axis_index("s") * NC + lax.axis_index("c")
    base = wid * B_PER_W

    pltpu.sync_copy(user_hbm.at[pl.ds(base, B_PER_W)], uidx_s)
    pltpu.sync_copy(item_hbm.at[pl.ds(base, B_PER_W)], iidx_s)

    copies = []
    for f in range(FACTORS):
        copies.append(
            pltpu.async_copy(utab_hbm.at[f].at[uidx_s], ug_v.at[f], sem_g))
        copies.append(
            pltpu.async_copy(itab_hbm.at[f].at[iidx_s], ig_v.at[f], sem_g))
    for c in copies:
        c.wait()

    @pl.loop(0, B_PER_W, step=L)
    def _(g):
        acc = jnp.zeros((L,), jnp.float32)
        for f in range(FACTORS):
            uu = ug_v.at[f][pl.ds(g, L)]
            vv = ig_v.at[f][pl.ds(g, L)]
            acc = acc + uu * vv
        out_v[pl.ds(g, L)] = acc

    pltpu.async_copy(out_v, out_hbm.at[pl.ds(base, B_PER_W)], sem_o).wait()


@jax.jit
def kernel(user, item, user_table, item_table):
    mesh = plsc.VectorSubcoreMesh(core_axis_name="c", subcore_axis_name="s")
    run = pl.kernel(
        _mf_dot_kernel,
        out_type=jax.ShapeDtypeStruct((BATCH,), jnp.float32),
        mesh=mesh,
        scratch_types=[
            pltpu.VMEM((B_PER_W,), jnp.int32),
            pltpu.VMEM((B_PER_W,), jnp.int32),
            pltpu.VMEM((FACTORS, B_PER_W), jnp.float32),
            pltpu.VMEM((FACTORS, B_PER_W), jnp.float32),
            pltpu.VMEM((B_PER_W,), jnp.float32),
            pltpu.SemaphoreType.DMA,
            pltpu.SemaphoreType.DMA,
        ],
        compiler_params=_make_compiler_params(),
    )
    return run(user.astype(jnp.int32), item.astype(jnp.int32),
               user_table.T, item_table.T)


# speedup vs baseline: 20.4158x; 20.4158x over previous
import dataclasses

import jax
import jax.numpy as jnp
from jax import lax
from jax.experimental import pallas as pl
from jax.experimental.pallas import tpu as pltpu
from jax.experimental.pallas import tpu_sc as plsc

NC = 2
NS = 16
L = 16
NW = NC * NS

BATCH = 16384
FACTORS = 32
B_PER_W = BATCH // NW
E_PER_CHUNK = 4
N_CHUNKS = B_PER_W // E_PER_CHUNK
WIN = 128


def _make_compiler_params():
    cp = pltpu.CompilerParams()
    fields = pltpu.CompilerParams.__dataclass_fields__
    if "needs_layout_passes" in fields:
        cp = dataclasses.replace(cp, needs_layout_passes=False)
    if "use_tc_tiling_on_sc" in fields:
        cp = dataclasses.replace(cp, use_tc_tiling_on_sc=True)
    return cp


def _mf_dot_kernel(user_hbm, item_hbm, utab_hbm, itab_hbm, out_hbm,
                   uidx_s, iidx_s, ub0, vb0, ub1, vb1, pg_v, out_v,
                   sem0, sem1, sem_o):
    wid = lax.axis_index("s") * NC + lax.axis_index("c")
    base = wid * B_PER_W

    pltpu.sync_copy(user_hbm.at[pl.ds(base, B_PER_W)],
                    uidx_s.at[pl.ds(0, B_PER_W)])
    pltpu.sync_copy(item_hbm.at[pl.ds(base, B_PER_W)],
                    iidx_s.at[pl.ds(0, B_PER_W)])

    iota = lax.iota(jnp.int32, L)

    def fire(k, ub, vb, sem):
        uvec = uidx_s[pl.ds(k * E_PER_CHUNK, L)]
        ivec = iidx_s[pl.ds(k * E_PER_CHUNK, L)]
        for j in range(E_PER_CHUNK):
            ru = uvec[j]
            wu = pl.multiple_of((ru // WIN) * WIN, WIN)
            pltpu.async_copy(utab_hbm.at[:, pl.ds(wu, WIN)],
                             ub.at[pl.ds(j * FACTORS, FACTORS), :], sem)
            ri = ivec[j]
            wi = pl.multiple_of((ri // WIN) * WIN, WIN)
            pltpu.async_copy(itab_hbm.at[:, pl.ds(wi, WIN)],
                             vb.at[pl.ds(j * FACTORS, FACTORS), :], sem)

    def drain(ub, vb, sem):
        for j in range(E_PER_CHUNK):
            pltpu.make_async_copy(
                utab_hbm.at[:, pl.ds(0, WIN)],
                ub.at[pl.ds(j * FACTORS, FACTORS), :], sem).wait()
            pltpu.make_async_copy(
                itab_hbm.at[:, pl.ds(0, WIN)],
                vb.at[pl.ds(j * FACTORS, FACTORS), :], sem).wait()

    def extract(k, ub, vb):
        uvec = uidx_s[pl.ds(k * E_PER_CHUNK, L)]
        ivec = iidx_s[pl.ds(k * E_PER_CHUNK, L)]
        for j in range(E_PER_CHUNK):
            e = k * E_PER_CHUNK + j
            ru = uvec[j]
            cu = jnp.full((L,), ru - (ru // WIN) * WIN, jnp.int32)
            ri = ivec[j]
            ci = jnp.full((L,), ri - (ri // WIN) * WIN, jnp.int32)
            ev = jnp.full((L,), e, jnp.int32)
            for half in (0, L):
                rows = iota + (j * FACTORS + half)
                uu = plsc.load_gather(ub, [rows, cu])
                vv = plsc.load_gather(vb, [rows, ci])
                plsc.store_scatter(pg_v, [iota + half, ev], uu * vv)

    fire(0, ub0, vb0, sem0)
    fire(1, ub1, vb1, sem1)

    @pl.loop(0, N_CHUNKS, step=2)
    def _(k):
        drain(ub0, vb0, sem0)
        extract(k, ub0, vb0)

        @pl.when(k + 2 < N_CHUNKS)
        def _():
            fire(k + 2, ub0, vb0, sem0)

        drain(ub1, vb1, sem1)
        extract(k + 1, ub1, vb1)

        @pl.when(k + 3 < N_CHUNKS)
        def _():
            fire(k + 3, ub1, vb1, sem1)

    @pl.loop(0, B_PER_W, step=L)
    def _(g):
        cols = iota + g
        acc = jnp.zeros((L,), jnp.float32)
        for f in range(FACTORS):
            acc = acc + plsc.load_gather(pg_v, [jnp.full((L,), f, jnp.int32),
                                                cols])
        out_v[pl.ds(g, L)] = acc

    pltpu.async_copy(out_v, out_hbm.at[pl.ds(base, B_PER_W)], sem_o).wait()


@jax.jit
def kernel(user, item, user_table, item_table):
    mesh = plsc.VectorSubcoreMesh(core_axis_name="c", subcore_axis_name="s")
    buf = pltpu.VMEM((E_PER_CHUNK * FACTORS, WIN), jnp.float32)
    run = pl.kernel(
        _mf_dot_kernel,
        out_type=jax.ShapeDtypeStruct((BATCH,), jnp.float32),
        mesh=mesh,
        scratch_types=[
            pltpu.VMEM((B_PER_W + L,), jnp.int32),
            pltpu.VMEM((B_PER_W + L,), jnp.int32),
            buf, buf, buf, buf,
            pltpu.VMEM((FACTORS, B_PER_W), jnp.float32),
            pltpu.VMEM((B_PER_W,), jnp.float32),
            pltpu.SemaphoreType.DMA,
            pltpu.SemaphoreType.DMA,
            pltpu.SemaphoreType.DMA,
        ],
        compiler_params=_make_compiler_params(),
    )
    return run(user.astype(jnp.int32), item.astype(jnp.int32),
               user_table.T, item_table.T)


# speedup vs baseline: 22.2116x; 1.0880x over previous
import dataclasses

import jax
import jax.numpy as jnp
from jax import lax
from jax.experimental import pallas as pl
from jax.experimental.pallas import tpu as pltpu
from jax.experimental.pallas import tpu_sc as plsc

NC = 2
NS = 16
L = 16
NW = NC * NS

BATCH = 16384
FACTORS = 32
B_PER_W = BATCH // NW
E_PER_CHUNK = 4
N_CHUNKS = B_PER_W // E_PER_CHUNK
WIN = 128


def _make_compiler_params():
    cp = pltpu.CompilerParams()
    fields = pltpu.CompilerParams.__dataclass_fields__
    if "needs_layout_passes" in fields:
        cp = dataclasses.replace(cp, needs_layout_passes=False)
    if "use_tc_tiling_on_sc" in fields:
        cp = dataclasses.replace(cp, use_tc_tiling_on_sc=True)
    return cp


def _mf_dot_kernel(user_hbm, item_hbm, utab_hbm, itab_hbm, out_hbm,
                   uidx_s, iidx_s, ub0, vb0, ub1, vb1, ub2, vb2, pg_v,
                   out_v, sem0, sem1, sem2, sem_o):
    wid = lax.axis_index("s") * NC + lax.axis_index("c")
    base = wid * B_PER_W

    pltpu.sync_copy(user_hbm.at[pl.ds(base, B_PER_W)],
                    uidx_s.at[pl.ds(0, B_PER_W)])
    pltpu.sync_copy(item_hbm.at[pl.ds(base, B_PER_W)],
                    iidx_s.at[pl.ds(0, B_PER_W)])

    iota = lax.iota(jnp.int32, L)

    def fire(k, ub, vb, sem):
        uvec = uidx_s[pl.ds(k * E_PER_CHUNK, L)]
        ivec = iidx_s[pl.ds(k * E_PER_CHUNK, L)]
        for j in range(E_PER_CHUNK):
            ru = uvec[j]
            wu = pl.multiple_of((ru // WIN) * WIN, WIN)
            pltpu.async_copy(utab_hbm.at[:, pl.ds(wu, WIN)],
                             ub.at[pl.ds(j * FACTORS, FACTORS), :], sem)
            ri = ivec[j]
            wi = pl.multiple_of((ri // WIN) * WIN, WIN)
            pltpu.async_copy(itab_hbm.at[:, pl.ds(wi, WIN)],
                             vb.at[pl.ds(j * FACTORS, FACTORS), :], sem)

    def drain(ub, vb, sem):
        for j in range(E_PER_CHUNK):
            pltpu.make_async_copy(
                utab_hbm.at[:, pl.ds(0, WIN)],
                ub.at[pl.ds(j * FACTORS, FACTORS), :], sem).wait()
            pltpu.make_async_copy(
                itab_hbm.at[:, pl.ds(0, WIN)],
                vb.at[pl.ds(j * FACTORS, FACTORS), :], sem).wait()

    def extract(k, ub, vb):
        uvec = uidx_s[pl.ds(k * E_PER_CHUNK, L)]
        ivec = iidx_s[pl.ds(k * E_PER_CHUNK, L)]
        for j in range(E_PER_CHUNK):
            e = k * E_PER_CHUNK + j
            ru = uvec[j]
            cu = jnp.full((L,), ru - (ru // WIN) * WIN, jnp.int32)
            ri = ivec[j]
            ci = jnp.full((L,), ri - (ri // WIN) * WIN, jnp.int32)
            ev = jnp.full((L,), e, jnp.int32)
            for half in (0, L):
                rows = iota + (j * FACTORS + half)
                uu = plsc.load_gather(ub, [rows, cu])
                vv = plsc.load_gather(vb, [rows, ci])
                plsc.store_scatter(pg_v, [iota + half, ev], uu * vv)

    fire(0, ub0, vb0, sem0)
    fire(1, ub1, vb1, sem1)
    fire(2, ub2, vb2, sem2)

    @pl.loop(0, N_CHUNKS + 1, step=3)
    def _(k):
        for i, (ub, vb, sem) in enumerate(((ub0, vb0, sem0),
                                           (ub1, vb1, sem1),
                                           (ub2, vb2, sem2))):
            c = k + i

            @pl.when(c < N_CHUNKS)
            def _():
                drain(ub, vb, sem)
                extract(c, ub, vb)

            @pl.when(c + 3 < N_CHUNKS)
            def _():
                fire(c + 3, ub, vb, sem)

    @pl.loop(0, B_PER_W, step=L)
    def _(g):
        cols = iota + g
        acc = jnp.zeros((L,), jnp.float32)
        for f in range(FACTORS):
            acc = acc + plsc.load_gather(pg_v, [jnp.full((L,), f, jnp.int32),
                                                cols])
        out_v[pl.ds(g, L)] = acc

    pltpu.async_copy(out_v, out_hbm.at[pl.ds(base, B_PER_W)], sem_o).wait()


@jax.jit
def kernel(user, item, user_table, item_table):
    mesh = plsc.VectorSubcoreMesh(core_axis_name="c", subcore_axis_name="s")
    buf = pltpu.VMEM((E_PER_CHUNK * FACTORS, WIN), jnp.float32)
    run = pl.kernel(
        _mf_dot_kernel,
        out_type=jax.ShapeDtypeStruct((BATCH,), jnp.float32),
        mesh=mesh,
        scratch_types=[
            pltpu.VMEM((B_PER_W + L,), jnp.int32),
            pltpu.VMEM((B_PER_W + L,), jnp.int32),
            buf, buf, buf, buf, buf, buf,
            pltpu.VMEM((FACTORS, B_PER_W), jnp.float32),
            pltpu.VMEM((B_PER_W,), jnp.float32),
            pltpu.SemaphoreType.DMA,
            pltpu.SemaphoreType.DMA,
            pltpu.SemaphoreType.DMA,
            pltpu.SemaphoreType.DMA,
        ],
        compiler_params=_make_compiler_params(),
    )
    return run(user.astype(jnp.int32), item.astype(jnp.int32),
               user_table.T, item_table.T)


# speedup vs baseline: 24.8812x; 1.1202x over previous
import dataclasses

import jax
import jax.numpy as jnp
from jax import lax
from jax.experimental import pallas as pl
from jax.experimental.pallas import tpu as pltpu
from jax.experimental.pallas import tpu_sc as plsc

NC = 2
NS = 16
L = 16
NW = NC * NS

BATCH = 16384
FACTORS = 32
B_PER_W = BATCH // NW
E_PER_CHUNK = 4
N_CHUNKS = B_PER_W // E_PER_CHUNK
WIN = 128


def _make_compiler_params():
    cp = pltpu.CompilerParams()
    fields = pltpu.CompilerParams.__dataclass_fields__
    if "needs_layout_passes" in fields:
        cp = dataclasses.replace(cp, needs_layout_passes=False)
    if "use_tc_tiling_on_sc" in fields:
        cp = dataclasses.replace(cp, use_tc_tiling_on_sc=True)
    return cp


def _mf_dot_kernel(user_hbm, item_hbm, utab_hbm, itab_hbm, out_hbm,
                   uidx_s, iidx_s, ub0, vb0, ub1, vb1, ub2, vb2, pg_v,
                   out_v, sem0, sem1, sem2, sem_o):
    wid = lax.axis_index("s") * NC + lax.axis_index("c")
    base = wid * B_PER_W

    pltpu.sync_copy(user_hbm.at[pl.ds(base, B_PER_W)],
                    uidx_s.at[pl.ds(0, B_PER_W)])
    pltpu.sync_copy(item_hbm.at[pl.ds(base, B_PER_W)],
                    iidx_s.at[pl.ds(0, B_PER_W)])

    iota = lax.iota(jnp.int32, L)
    fhalf0 = iota
    fhalf1 = iota + L

    def fire(k, ub, vb, sem):
        uvec = uidx_s[pl.ds(k * E_PER_CHUNK, L)]
        ivec = iidx_s[pl.ds(k * E_PER_CHUNK, L)]
        for j in range(E_PER_CHUNK):
            ru = uvec[j]
            wu = pl.multiple_of((ru // WIN) * WIN, WIN)
            pltpu.async_copy(utab_hbm.at[:, pl.ds(wu, WIN)],
                             ub.at[pl.ds(j * FACTORS, FACTORS), :], sem)
            ri = ivec[j]
            wi = pl.multiple_of((ri // WIN) * WIN, WIN)
            pltpu.async_copy(itab_hbm.at[fhalf0, pl.ds(wi, WIN)],
                             vb.at[pl.ds(j * FACTORS, L), :], sem)
            pltpu.async_copy(itab_hbm.at[fhalf1, pl.ds(wi, WIN)],
                             vb.at[pl.ds(j * FACTORS + L, L), :], sem)

    def drain(ub, vb, sem):
        for j in range(E_PER_CHUNK):
            pltpu.make_async_copy(
                utab_hbm.at[:, pl.ds(0, WIN)],
                ub.at[pl.ds(j * FACTORS, FACTORS), :], sem).wait()
            pltpu.make_async_copy(
                itab_hbm.at[:, pl.ds(0, WIN)],
                vb.at[pl.ds(j * FACTORS, FACTORS), :], sem).wait()

    def extract(k, ub, vb):
        uvec = uidx_s[pl.ds(k * E_PER_CHUNK, L)]
        ivec = iidx_s[pl.ds(k * E_PER_CHUNK, L)]
        for j in range(E_PER_CHUNK):
            e = k * E_PER_CHUNK + j
            ru = uvec[j]
            cu = jnp.full((L,), ru - (ru // WIN) * WIN, jnp.int32)
            ri = ivec[j]
            ci = jnp.full((L,), ri - (ri // WIN) * WIN, jnp.int32)
            ev = jnp.full((L,), e, jnp.int32)
            for half in (0, L):
                rows = iota + (j * FACTORS + half)
                uu = plsc.load_gather(ub, [rows, cu])
                vv = plsc.load_gather(vb, [rows, ci])
                plsc.store_scatter(pg_v, [iota + half, ev], uu * vv)

    fire(0, ub0, vb0, sem0)
    fire(1, ub1, vb1, sem1)
    fire(2, ub2, vb2, sem2)

    @pl.loop(0, N_CHUNKS + 1, step=3)
    def _(k):
        for i, (ub, vb, sem) in enumerate(((ub0, vb0, sem0),
                                           (ub1, vb1, sem1),
                                           (ub2, vb2, sem2))):
            c = k + i

            @pl.when(c < N_CHUNKS)
            def _():
                drain(ub, vb, sem)
                extract(c, ub, vb)

            @pl.when(c + 3 < N_CHUNKS)
            def _():
                fire(c + 3, ub, vb, sem)

    @pl.loop(0, B_PER_W, step=L)
    def _(g):
        cols = iota + g
        acc = jnp.zeros((L,), jnp.float32)
        for f in range(FACTORS):
            acc = acc + plsc.load_gather(pg_v, [jnp.full((L,), f, jnp.int32),
                                                cols])
        out_v[pl.ds(g, L)] = acc

    pltpu.async_copy(out_v, out_hbm.at[pl.ds(base, B_PER_W)], sem_o).wait()


@jax.jit
def kernel(user, item, user_table, item_table):
    mesh = plsc.VectorSubcoreMesh(core_axis_name="c", subcore_axis_name="s")
    buf = pltpu.VMEM((E_PER_CHUNK * FACTORS, WIN), jnp.float32)
    run = pl.kernel(
        _mf_dot_kernel,
        out_type=jax.ShapeDtypeStruct((BATCH,), jnp.float32),
        mesh=mesh,
        scratch_types=[
            pltpu.VMEM((B_PER_W + L,), jnp.int32),
            pltpu.VMEM((B_PER_W + L,), jnp.int32),
            buf, buf, buf, buf, buf, buf,
            pltpu.VMEM((FACTORS, B_PER_W), jnp.float32),
            pltpu.VMEM((B_PER_W,), jnp.float32),
            pltpu.SemaphoreType.DMA,
            pltpu.SemaphoreType.DMA,
            pltpu.SemaphoreType.DMA,
            pltpu.SemaphoreType.DMA,
        ],
        compiler_params=_make_compiler_params(),
    )
    return run(user.astype(jnp.int32), item.astype(jnp.int32),
               user_table.T, item_table.T)
